# baseline (speedup 1.0000x reference)
import jax
import jax.numpy as jnp
from jax.experimental import pallas as pl


def _copy_kernel(tok_ref, out_ref):
    out_ref[...] = tok_ref[...]


@jax.jit
def _run(tokens2):
    n, d = tokens2.shape
    blk = 4608
    return pl.pallas_call(
        _copy_kernel,
        grid=(n // blk,),
        in_specs=[pl.BlockSpec((blk, d), lambda i: (i, 0))],
        out_specs=pl.BlockSpec((blk, d), lambda i: (i, 0)),
        out_shape=jax.ShapeDtypeStruct(tokens2.shape, tokens2.dtype),
    )(tokens2)


def kernel(tokens, channel_embeddings, timestamps, patch_size, input_res):
    b, h, w, t, c, d = tokens.shape
    out = _run(tokens.reshape(b * h * w * t * c, d))
    return out.reshape(b, h, w, t, c, d)


# speedup vs baseline: 2.2497x; 2.2497x over previous
import jax
import jax.numpy as jnp
import numpy as np
from jax.experimental import pallas as pl
from jax.experimental.pallas import tpu as pltpu

EMBED_SIZE = 768
D_TYPE = EMBED_SIZE // 4
MAX_SEQ = 24
BASE_GSD = 10.0

NBUF = 4


def _sincos_1d(pos, dim):
    omega = 1.0 / (10000.0 ** (jnp.arange(dim // 2, dtype=jnp.float32) / (dim / 2.0)))
    out = pos.astype(jnp.float32)[:, None] * omega[None, :]
    return jnp.concatenate([jnp.sin(out), jnp.cos(out)], axis=-1)


def _month_table(dim):
    angles = jnp.arange(0, 13, dtype=jnp.float32) / (12.0 / (2.0 * np.pi))
    ang = jnp.stack([angles] * (dim // 2), axis=-1)
    return jnp.concatenate([jnp.sin(ang)[:-1], jnp.cos(ang)[:-1]], axis=-1)


def _emb_from_grid_1d(pos, dim):
    omega = 1.0 / (10000.0 ** (jnp.arange(dim // 2, dtype=jnp.float32) / (dim / 2.0)))
    flat = pos.reshape(pos.shape[0], -1)
    out = flat[..., None] * omega[None, None, :]
    return jnp.concatenate([jnp.sin(out), jnp.cos(out)], axis=-1)


def _spatial_table(grid_size, res, dim):
    coords = jnp.arange(grid_size, dtype=jnp.float32)
    gw, gh = jnp.meshgrid(coords, coords, indexing='xy')
    grid = jnp.stack([gw, gh], axis=0)
    grid = grid[None, :, :, :] * res[:, None, None, None]
    emb_h = _emb_from_grid_1d(grid[:, 0], dim // 2)
    emb_w = _emb_from_grid_1d(grid[:, 1], dim // 2)
    return jnp.concatenate([emb_h, emb_w], axis=-1)


def _add_kernel(months_ref, a_ref, s_ref, mt_ref, tok_hbm, out_hbm,
                in_buf, out_buf, ae_buf, sem_in, sem_out):
    i = pl.program_id(0)
    n = pl.num_programs(0)
    slot = jax.lax.rem(i, NBUF)

    @pl.when(i == 0)
    def _prologue():
        for bb in range(4):
            m_ids = months_ref[bb, 0]
            k12 = jax.lax.broadcasted_iota(jnp.int32, (12, 12), 1)
            oh = (m_ids[:, None] == k12).astype(jnp.float32)
            mo = jnp.dot(oh, mt_ref[...], preferred_element_type=jnp.float32)
            r36 = jax.lax.broadcasted_iota(jnp.int32, (36, 12), 0) // 3
            t36 = jax.lax.broadcasted_iota(jnp.int32, (36, 12), 1)
            rep = (r36 == t36).astype(jnp.float32)
            mo36 = jnp.dot(rep, mo, preferred_element_type=jnp.float32)
            ae_buf[bb] = a_ref[...] + mo36
        for j in range(NBUF - 1):
            pltpu.make_async_copy(tok_hbm.at[j], in_buf.at[j], sem_in.at[j]).start()

    nxt = i + NBUF - 1

    @pl.when(nxt < n)
    def _issue_in():
        nslot = jax.lax.rem(nxt, NBUF)
        pltpu.make_async_copy(tok_hbm.at[nxt], in_buf.at[nslot], sem_in.at[nslot]).start()

    pltpu.make_async_copy(tok_hbm.at[i], in_buf.at[slot], sem_in.at[slot]).wait()

    @pl.when(i >= NBUF)
    def _wait_out():
        pltpu.make_async_copy(out_buf.at[slot], out_hbm.at[i], sem_out.at[slot]).wait()

    bsel = i // 16
    ae = ae_buf[bsel]
    s = s_ref[i]
    out_buf[slot] = in_buf[slot] + ae[None, :, :] + s[:, None, :]

    pltpu.make_async_copy(out_buf.at[slot], out_hbm.at[i], sem_out.at[slot]).start()

    @pl.when(i == n - 1)
    def _drain():
        for j in range(NBUF):
            k = n - NBUF + j
            kslot = k % NBUF
            pltpu.make_async_copy(out_buf.at[kslot], out_hbm.at[k], sem_out.at[kslot]).wait()


@jax.jit
def _run(tokens4, a_table, s_table, months3, mtable):
    n, w, r, d = tokens4.shape
    return pl.pallas_call(
        _add_kernel,
        grid=(n,),
        in_specs=[
            pl.BlockSpec(memory_space=pltpu.MemorySpace.VMEM),
            pl.BlockSpec(memory_space=pltpu.MemorySpace.VMEM),
            pl.BlockSpec(memory_space=pltpu.MemorySpace.VMEM),
            pl.BlockSpec(memory_space=pltpu.MemorySpace.VMEM),
            pl.BlockSpec(memory_space=pltpu.MemorySpace.HBM),
        ],
        out_specs=pl.BlockSpec(memory_space=pltpu.MemorySpace.HBM),
        out_shape=jax.ShapeDtypeStruct(tokens4.shape, tokens4.dtype),
        scratch_shapes=[
            pltpu.VMEM((NBUF, w, r, d), jnp.float32),
            pltpu.VMEM((NBUF, w, r, d), jnp.float32),
            pltpu.VMEM((4, r, d), jnp.float32),
            pltpu.SemaphoreType.DMA((NBUF,)),
            pltpu.SemaphoreType.DMA((NBUF,)),
        ],
    )(months3, a_table, s_table, mtable, tokens4)


def kernel(tokens, channel_embeddings, timestamps, patch_size, input_res):
    b, h, w, t, c, d = tokens.shape
    dt = d // 4

    pos = _sincos_1d(jnp.arange(MAX_SEQ), dt)[:t]
    a_table = jnp.concatenate(
        [jnp.broadcast_to(channel_embeddings[None, :, :], (t, c, dt)),
         jnp.broadcast_to(pos[:, None, :], (t, c, dt)),
         jnp.zeros((t, c, 2 * dt), dtype=jnp.float32)],
        axis=-1).reshape(t * c, d)

    gsd_ratio = (jnp.asarray(input_res).astype(jnp.float32)
                 * jnp.asarray(patch_size).astype(jnp.float32) / BASE_GSD)
    spatial = _spatial_table(h, jnp.ones((b,), dtype=jnp.float32) * gsd_ratio, dt)
    spatial = spatial.reshape(b, h, w, dt)
    s_table = jnp.concatenate(
        [jnp.zeros((b, h, w, 3 * dt), dtype=jnp.float32), spatial],
        axis=-1).reshape(b * h, w, d)

    mtable = jnp.concatenate(
        [jnp.zeros((12, 2 * dt), dtype=jnp.float32), _month_table(dt),
         jnp.zeros((12, dt), dtype=jnp.float32)], axis=-1)

    months3 = timestamps[:, 1, :].astype(jnp.int32).reshape(b, 1, t)

    tokens4 = tokens.reshape(b * h, w, t * c, d)
    out = _run(tokens4, a_table, s_table, months3, mtable)
    return out.reshape(b, h, w, t, c, d)


# speedup vs baseline: 2.2517x; 1.0009x over previous
import jax
import jax.numpy as jnp
import numpy as np
from jax.experimental import pallas as pl
from jax.experimental.pallas import tpu as pltpu

EMBED_SIZE = 768
D_TYPE = EMBED_SIZE // 4
MAX_SEQ = 24
BASE_GSD = 10.0

NBUF = 4
NSTR = 4
SROW = 4


def _sincos_1d(pos, dim):
    omega = 1.0 / (10000.0 ** (jnp.arange(dim // 2, dtype=jnp.float32) / (dim / 2.0)))
    out = pos.astype(jnp.float32)[:, None] * omega[None, :]
    return jnp.concatenate([jnp.sin(out), jnp.cos(out)], axis=-1)


def _month_table(dim):
    angles = jnp.arange(0, 13, dtype=jnp.float32) / (12.0 / (2.0 * np.pi))
    ang = jnp.stack([angles] * (dim // 2), axis=-1)
    return jnp.concatenate([jnp.sin(ang)[:-1], jnp.cos(ang)[:-1]], axis=-1)


def _emb_from_grid_1d(pos, dim):
    omega = 1.0 / (10000.0 ** (jnp.arange(dim // 2, dtype=jnp.float32) / (dim / 2.0)))
    flat = pos.reshape(pos.shape[0], -1)
    out = flat[..., None] * omega[None, None, :]
    return jnp.concatenate([jnp.sin(out), jnp.cos(out)], axis=-1)


def _spatial_table(grid_size, res, dim):
    coords = jnp.arange(grid_size, dtype=jnp.float32)
    gw, gh = jnp.meshgrid(coords, coords, indexing='xy')
    grid = jnp.stack([gw, gh], axis=0)
    grid = grid[None, :, :, :] * res[:, None, None, None]
    emb_h = _emb_from_grid_1d(grid[:, 0], dim // 2)
    emb_w = _emb_from_grid_1d(grid[:, 1], dim // 2)
    return jnp.concatenate([emb_h, emb_w], axis=-1)


def _add_kernel(months_ref, a_ref, s_ref, mt_ref, tok_hbm, out_hbm,
                in_buf, out_buf, ae_buf, sem_in, sem_out):
    i = pl.program_id(0)
    n = pl.num_programs(0)
    slot = jax.lax.rem(i, NBUF)

    @pl.when(i == 0)
    def _prologue():
        for bb in range(4):
            m_ids = months_ref[bb, 0]
            k12 = jax.lax.broadcasted_iota(jnp.int32, (12, 12), 1)
            oh = (m_ids[:, None] == k12).astype(jnp.float32)
            mo = jnp.dot(oh, mt_ref[...], preferred_element_type=jnp.float32)
            r36 = jax.lax.broadcasted_iota(jnp.int32, (36, 12), 0) // 3
            t36 = jax.lax.broadcasted_iota(jnp.int32, (36, 12), 1)
            rep = (r36 == t36).astype(jnp.float32)
            mo36 = jnp.dot(rep, mo, preferred_element_type=jnp.float32)
            ae_buf[bb] = a_ref[...] + mo36
        for j in range(NBUF - 1):
            for st in range(NSTR):
                pltpu.make_async_copy(tok_hbm.at[j, pl.ds(st * SROW, SROW)],
                                      in_buf.at[j, pl.ds(st * SROW, SROW)],
                                      sem_in.at[j, st]).start()

    nxt = i + NBUF - 1

    @pl.when(nxt < n)
    def _issue_in():
        nslot = jax.lax.rem(nxt, NBUF)
        for st in range(NSTR):
            pltpu.make_async_copy(tok_hbm.at[nxt, pl.ds(st * SROW, SROW)],
                                  in_buf.at[nslot, pl.ds(st * SROW, SROW)],
                                  sem_in.at[nslot, st]).start()

    for st in range(NSTR):
        pltpu.make_async_copy(tok_hbm.at[i, pl.ds(st * SROW, SROW)],
                              in_buf.at[slot, pl.ds(st * SROW, SROW)],
                              sem_in.at[slot, st]).wait()

    @pl.when(i >= NBUF)
    def _wait_out():
        for st in range(NSTR):
            pltpu.make_async_copy(out_buf.at[slot, pl.ds(st * SROW, SROW)],
                                  out_hbm.at[i, pl.ds(st * SROW, SROW)],
                                  sem_out.at[slot, st]).wait()

    bsel = i // 16
    ae = ae_buf[bsel]
    s = s_ref[i]
    out_buf[slot] = in_buf[slot] + ae[None, :, :] + s[:, None, :]

    for st in range(NSTR):
        pltpu.make_async_copy(out_buf.at[slot, pl.ds(st * SROW, SROW)],
                              out_hbm.at[i, pl.ds(st * SROW, SROW)],
                              sem_out.at[slot, st]).start()

    @pl.when(i == n - 1)
    def _drain():
        for j in range(NBUF):
            k = n - NBUF + j
            kslot = k % NBUF
            for st in range(NSTR):
                pltpu.make_async_copy(out_buf.at[kslot, pl.ds(st * SROW, SROW)],
                                      out_hbm.at[k, pl.ds(st * SROW, SROW)],
                                      sem_out.at[kslot, st]).wait()


@jax.jit
def _run(tokens4, a_table, s_table, months3, mtable):
    n, w, r, d = tokens4.shape
    return pl.pallas_call(
        _add_kernel,
        grid=(n,),
        in_specs=[
            pl.BlockSpec(memory_space=pltpu.MemorySpace.VMEM),
            pl.BlockSpec(memory_space=pltpu.MemorySpace.VMEM),
            pl.BlockSpec(memory_space=pltpu.MemorySpace.VMEM),
            pl.BlockSpec(memory_space=pltpu.MemorySpace.VMEM),
            pl.BlockSpec(memory_space=pltpu.MemorySpace.HBM),
        ],
        out_specs=pl.BlockSpec(memory_space=pltpu.MemorySpace.HBM),
        out_shape=jax.ShapeDtypeStruct(tokens4.shape, tokens4.dtype),
        scratch_shapes=[
            pltpu.VMEM((NBUF, w, r, d), jnp.float32),
            pltpu.VMEM((NBUF, w, r, d), jnp.float32),
            pltpu.VMEM((4, r, d), jnp.float32),
            pltpu.SemaphoreType.DMA((NBUF, NSTR)),
            pltpu.SemaphoreType.DMA((NBUF, NSTR)),
        ],
    )(months3, a_table, s_table, mtable, tokens4)


def kernel(tokens, channel_embeddings, timestamps, patch_size, input_res):
    b, h, w, t, c, d = tokens.shape
    dt = d // 4

    pos = _sincos_1d(jnp.arange(MAX_SEQ), dt)[:t]
    a_table = jnp.concatenate(
        [jnp.broadcast_to(channel_embeddings[None, :, :], (t, c, dt)),
         jnp.broadcast_to(pos[:, None, :], (t, c, dt)),
         jnp.zeros((t, c, 2 * dt), dtype=jnp.float32)],
        axis=-1).reshape(t * c, d)

    gsd_ratio = (jnp.asarray(input_res).astype(jnp.float32)
                 * jnp.asarray(patch_size).astype(jnp.float32) / BASE_GSD)
    spatial = _spatial_table(h, jnp.ones((b,), dtype=jnp.float32) * gsd_ratio, dt)
    spatial = spatial.reshape(b, h, w, dt)
    s_table = jnp.concatenate(
        [jnp.zeros((b, h, w, 3 * dt), dtype=jnp.float32), spatial],
        axis=-1).reshape(b * h, w, d)

    mtable = jnp.concatenate(
        [jnp.zeros((12, 2 * dt), dtype=jnp.float32), _month_table(dt),
         jnp.zeros((12, dt), dtype=jnp.float32)], axis=-1)

    months3 = timestamps[:, 1, :].astype(jnp.int32).reshape(b, 1, t)

    tokens4 = tokens.reshape(b * h, w, t * c, d)
    out = _run(tokens4, a_table, s_table, months3, mtable)
    return out.reshape(b, h, w, t, c, d)


# speedup vs baseline: 7.2482x; 3.2190x over previous
import jax
import jax.numpy as jnp
import numpy as np
from jax.experimental import pallas as pl
from jax.experimental.pallas import tpu as pltpu

EMBED_SIZE = 768
D_TYPE = EMBED_SIZE // 4
MAX_SEQ = 24
BASE_GSD = 10.0

NBUF = 4
NSTR = 4
SROW = 9


def _sincos_1d(pos, dim):
    omega = 1.0 / (10000.0 ** (jnp.arange(dim // 2, dtype=jnp.float32) / (dim / 2.0)))
    out = pos.astype(jnp.float32)[:, None] * omega[None, :]
    return jnp.concatenate([jnp.sin(out), jnp.cos(out)], axis=-1)


def _month_table(dim):
    angles = jnp.arange(0, 13, dtype=jnp.float32) / (12.0 / (2.0 * np.pi))
    ang = jnp.stack([angles] * (dim // 2), axis=-1)
    return jnp.concatenate([jnp.sin(ang)[:-1], jnp.cos(ang)[:-1]], axis=-1)


def _emb_from_grid_1d(pos, dim):
    omega = 1.0 / (10000.0 ** (jnp.arange(dim // 2, dtype=jnp.float32) / (dim / 2.0)))
    flat = pos.reshape(pos.shape[0], -1)
    out = flat[..., None] * omega[None, None, :]
    return jnp.concatenate([jnp.sin(out), jnp.cos(out)], axis=-1)


def _spatial_table(grid_size, res, dim):
    coords = jnp.arange(grid_size, dtype=jnp.float32)
    gw, gh = jnp.meshgrid(coords, coords, indexing='xy')
    grid = jnp.stack([gw, gh], axis=0)
    grid = grid[None, :, :, :] * res[:, None, None, None]
    emb_h = _emb_from_grid_1d(grid[:, 0], dim // 2)
    emb_w = _emb_from_grid_1d(grid[:, 1], dim // 2)
    return jnp.concatenate([emb_h, emb_w], axis=-1)


def _add_kernel(months_ref, a_ref, s_ref, mt_ref, tok_hbm, out_hbm,
                in_buf, out_buf, ae_buf, sem_in, sem_out):
    i = pl.program_id(0)
    n = pl.num_programs(0)
    slot = jax.lax.rem(i, NBUF)

    @pl.when(i == 0)
    def _prologue():
        for bb in range(4):
            m_ids = months_ref[bb, 0]
            k12 = jax.lax.broadcasted_iota(jnp.int32, (12, 12), 1)
            oh = (m_ids[:, None] == k12).astype(jnp.float32)
            mo = jnp.dot(oh, mt_ref[...], preferred_element_type=jnp.float32)
            r36 = jax.lax.broadcasted_iota(jnp.int32, (36, 12), 0) // 3
            t36 = jax.lax.broadcasted_iota(jnp.int32, (36, 12), 1)
            rep = (r36 == t36).astype(jnp.float32)
            mo36 = jnp.dot(rep, mo, preferred_element_type=jnp.float32)
            ae_buf[bb] = a_ref[...] + mo36
        for j in range(NBUF - 1):
            for st in range(NSTR):
                pltpu.make_async_copy(tok_hbm.at[j, pl.ds(st * SROW, SROW)],
                                      in_buf.at[j, pl.ds(st * SROW, SROW)],
                                      sem_in.at[j, st]).start()

    nxt = i + NBUF - 1

    @pl.when(nxt < n)
    def _issue_in():
        nslot = jax.lax.rem(nxt, NBUF)
        for st in range(NSTR):
            pltpu.make_async_copy(tok_hbm.at[nxt, pl.ds(st * SROW, SROW)],
                                  in_buf.at[nslot, pl.ds(st * SROW, SROW)],
                                  sem_in.at[nslot, st]).start()

    for st in range(NSTR):
        pltpu.make_async_copy(tok_hbm.at[i, pl.ds(st * SROW, SROW)],
                              in_buf.at[slot, pl.ds(st * SROW, SROW)],
                              sem_in.at[slot, st]).wait()

    @pl.when(i >= NBUF)
    def _wait_out():
        for st in range(NSTR):
            pltpu.make_async_copy(out_buf.at[slot, pl.ds(st * SROW, SROW)],
                                  out_hbm.at[i, pl.ds(st * SROW, SROW)],
                                  sem_out.at[slot, st]).wait()

    bsel = i // 16
    ae = ae_buf[bsel]
    s = s_ref[i]
    out_buf[slot] = in_buf[slot] + ae[:, None, :] + s[None, :, :]

    for st in range(NSTR):
        pltpu.make_async_copy(out_buf.at[slot, pl.ds(st * SROW, SROW)],
                              out_hbm.at[i, pl.ds(st * SROW, SROW)],
                              sem_out.at[slot, st]).start()

    @pl.when(i == n - 1)
    def _drain():
        for j in range(NBUF):
            k = n - NBUF + j
            kslot = k % NBUF
            for st in range(NSTR):
                pltpu.make_async_copy(out_buf.at[kslot, pl.ds(st * SROW, SROW)],
                                      out_hbm.at[k, pl.ds(st * SROW, SROW)],
                                      sem_out.at[kslot, st]).wait()


@jax.jit
def _run(tokens4, a_table, s_table, months3, mtable):
    n, r, w, d = tokens4.shape
    return pl.pallas_call(
        _add_kernel,
        grid=(n,),
        in_specs=[
            pl.BlockSpec(memory_space=pltpu.MemorySpace.VMEM),
            pl.BlockSpec(memory_space=pltpu.MemorySpace.VMEM),
            pl.BlockSpec(memory_space=pltpu.MemorySpace.VMEM),
            pl.BlockSpec(memory_space=pltpu.MemorySpace.VMEM),
            pl.BlockSpec(memory_space=pltpu.MemorySpace.HBM),
        ],
        out_specs=pl.BlockSpec(memory_space=pltpu.MemorySpace.HBM),
        out_shape=jax.ShapeDtypeStruct(tokens4.shape, tokens4.dtype),
        scratch_shapes=[
            pltpu.VMEM((NBUF, r, w, d), jnp.float32),
            pltpu.VMEM((NBUF, r, w, d), jnp.float32),
            pltpu.VMEM((4, r, d), jnp.float32),
            pltpu.SemaphoreType.DMA((NBUF, NSTR)),
            pltpu.SemaphoreType.DMA((NBUF, NSTR)),
        ],
    )(months3, a_table, s_table, mtable, tokens4)


def kernel(tokens, channel_embeddings, timestamps, patch_size, input_res):
    b, h, w, t, c, d = tokens.shape
    dt = d // 4

    pos = _sincos_1d(jnp.arange(MAX_SEQ), dt)[:t]
    a_table = jnp.concatenate(
        [jnp.broadcast_to(channel_embeddings[None, :, :], (t, c, dt)),
         jnp.broadcast_to(pos[:, None, :], (t, c, dt)),
         jnp.zeros((t, c, 2 * dt), dtype=jnp.float32)],
        axis=-1).reshape(t * c, d)

    gsd_ratio = (jnp.asarray(input_res).astype(jnp.float32)
                 * jnp.asarray(patch_size).astype(jnp.float32) / BASE_GSD)
    spatial = _spatial_table(h, jnp.ones((b,), dtype=jnp.float32) * gsd_ratio, dt)
    spatial = spatial.reshape(b, h, w, dt)
    s_table = jnp.concatenate(
        [jnp.zeros((b, h, w, 3 * dt), dtype=jnp.float32), spatial],
        axis=-1).reshape(b * h, w, d)

    mtable = jnp.concatenate(
        [jnp.zeros((12, 2 * dt), dtype=jnp.float32), _month_table(dt),
         jnp.zeros((12, dt), dtype=jnp.float32)], axis=-1)

    months3 = timestamps[:, 1, :].astype(jnp.int32).reshape(b, 1, t)

    tokens4 = jnp.transpose(tokens, (0, 1, 3, 4, 2, 5)).reshape(b * h, t * c, w, d)
    out = _run(tokens4, a_table, s_table, months3, mtable)
    return jnp.transpose(out.reshape(b, h, t, c, w, d), (0, 1, 4, 2, 3, 5))


# speedup vs baseline: 7.2623x; 1.0020x over previous
import jax
import jax.numpy as jnp
import numpy as np
from jax.experimental import pallas as pl
from jax.experimental.pallas import tpu as pltpu

EMBED_SIZE = 768
D_TYPE = EMBED_SIZE // 4
MAX_SEQ = 24
BASE_GSD = 10.0

NBUF = 6
NSTR = 2
SROW = 18


def _sincos_1d(pos, dim):
    omega = 1.0 / (10000.0 ** (jnp.arange(dim // 2, dtype=jnp.float32) / (dim / 2.0)))
    out = pos.astype(jnp.float32)[:, None] * omega[None, :]
    return jnp.concatenate([jnp.sin(out), jnp.cos(out)], axis=-1)


def _month_table(dim):
    angles = jnp.arange(0, 13, dtype=jnp.float32) / (12.0 / (2.0 * np.pi))
    ang = jnp.stack([angles] * (dim // 2), axis=-1)
    return jnp.concatenate([jnp.sin(ang)[:-1], jnp.cos(ang)[:-1]], axis=-1)


def _emb_from_grid_1d(pos, dim):
    omega = 1.0 / (10000.0 ** (jnp.arange(dim // 2, dtype=jnp.float32) / (dim / 2.0)))
    flat = pos.reshape(pos.shape[0], -1)
    out = flat[..., None] * omega[None, None, :]
    return jnp.concatenate([jnp.sin(out), jnp.cos(out)], axis=-1)


def _spatial_table(grid_size, res, dim):
    coords = jnp.arange(grid_size, dtype=jnp.float32)
    gw, gh = jnp.meshgrid(coords, coords, indexing='xy')
    grid = jnp.stack([gw, gh], axis=0)
    grid = grid[None, :, :, :] * res[:, None, None, None]
    emb_h = _emb_from_grid_1d(grid[:, 0], dim // 2)
    emb_w = _emb_from_grid_1d(grid[:, 1], dim // 2)
    return jnp.concatenate([emb_h, emb_w], axis=-1)


def _add_kernel(months_ref, a_ref, s_ref, mt_ref, tok_hbm, out_hbm,
                in_buf, out_buf, ae_buf, sem_in, sem_out):
    i = pl.program_id(0)
    n = pl.num_programs(0)
    slot = jax.lax.rem(i, NBUF)

    @pl.when(i == 0)
    def _prologue():
        for bb in range(4):
            m_ids = months_ref[bb, 0]
            k12 = jax.lax.broadcasted_iota(jnp.int32, (12, 12), 1)
            oh = (m_ids[:, None] == k12).astype(jnp.float32)
            mo = jnp.dot(oh, mt_ref[...], preferred_element_type=jnp.float32)
            r36 = jax.lax.broadcasted_iota(jnp.int32, (36, 12), 0) // 3
            t36 = jax.lax.broadcasted_iota(jnp.int32, (36, 12), 1)
            rep = (r36 == t36).astype(jnp.float32)
            mo36 = jnp.dot(rep, mo, preferred_element_type=jnp.float32)
            ae_buf[bb] = a_ref[...] + mo36
        for j in range(NBUF - 1):
            for st in range(NSTR):
                pltpu.make_async_copy(tok_hbm.at[j, pl.ds(st * SROW, SROW)],
                                      in_buf.at[j, pl.ds(st * SROW, SROW)],
                                      sem_in.at[j, st]).start()

    nxt = i + NBUF - 1

    @pl.when(nxt < n)
    def _issue_in():
        nslot = jax.lax.rem(nxt, NBUF)
        for st in range(NSTR):
            pltpu.make_async_copy(tok_hbm.at[nxt, pl.ds(st * SROW, SROW)],
                                  in_buf.at[nslot, pl.ds(st * SROW, SROW)],
                                  sem_in.at[nslot, st]).start()

    for st in range(NSTR):
        pltpu.make_async_copy(tok_hbm.at[i, pl.ds(st * SROW, SROW)],
                              in_buf.at[slot, pl.ds(st * SROW, SROW)],
                              sem_in.at[slot, st]).wait()

    @pl.when(i >= NBUF)
    def _wait_out():
        for st in range(NSTR):
            pltpu.make_async_copy(out_buf.at[slot, pl.ds(st * SROW, SROW)],
                                  out_hbm.at[i, pl.ds(st * SROW, SROW)],
                                  sem_out.at[slot, st]).wait()

    bsel = i // 16
    ae = ae_buf[bsel]
    s = s_ref[i]
    out_buf[slot] = in_buf[slot] + ae[:, None, :] + s[None, :, :]

    for st in range(NSTR):
        pltpu.make_async_copy(out_buf.at[slot, pl.ds(st * SROW, SROW)],
                              out_hbm.at[i, pl.ds(st * SROW, SROW)],
                              sem_out.at[slot, st]).start()

    @pl.when(i == n - 1)
    def _drain():
        for j in range(NBUF):
            k = n - NBUF + j
            kslot = k % NBUF
            for st in range(NSTR):
                pltpu.make_async_copy(out_buf.at[kslot, pl.ds(st * SROW, SROW)],
                                      out_hbm.at[k, pl.ds(st * SROW, SROW)],
                                      sem_out.at[kslot, st]).wait()


@jax.jit
def _run(tokens4, a_table, s_table, months3, mtable):
    n, r, w, d = tokens4.shape
    return pl.pallas_call(
        _add_kernel,
        grid=(n,),
        in_specs=[
            pl.BlockSpec(memory_space=pltpu.MemorySpace.VMEM),
            pl.BlockSpec(memory_space=pltpu.MemorySpace.VMEM),
            pl.BlockSpec(memory_space=pltpu.MemorySpace.VMEM),
            pl.BlockSpec(memory_space=pltpu.MemorySpace.VMEM),
            pl.BlockSpec(memory_space=pltpu.MemorySpace.HBM),
        ],
        out_specs=pl.BlockSpec(memory_space=pltpu.MemorySpace.HBM),
        out_shape=jax.ShapeDtypeStruct(tokens4.shape, tokens4.dtype),
        scratch_shapes=[
            pltpu.VMEM((NBUF, r, w, d), jnp.float32),
            pltpu.VMEM((NBUF, r, w, d), jnp.float32),
            pltpu.VMEM((4, r, d), jnp.float32),
            pltpu.SemaphoreType.DMA((NBUF, NSTR)),
            pltpu.SemaphoreType.DMA((NBUF, NSTR)),
        ],
    )(months3, a_table, s_table, mtable, tokens4)


def kernel(tokens, channel_embeddings, timestamps, patch_size, input_res):
    b, h, w, t, c, d = tokens.shape
    dt = d // 4

    pos = _sincos_1d(jnp.arange(MAX_SEQ), dt)[:t]
    a_table = jnp.concatenate(
        [jnp.broadcast_to(channel_embeddings[None, :, :], (t, c, dt)),
         jnp.broadcast_to(pos[:, None, :], (t, c, dt)),
         jnp.zeros((t, c, 2 * dt), dtype=jnp.float32)],
        axis=-1).reshape(t * c, d)

    gsd_ratio = (jnp.asarray(input_res).astype(jnp.float32)
                 * jnp.asarray(patch_size).astype(jnp.float32) / BASE_GSD)
    spatial = _spatial_table(h, jnp.ones((b,), dtype=jnp.float32) * gsd_ratio, dt)
    spatial = spatial.reshape(b, h, w, dt)
    s_table = jnp.concatenate(
        [jnp.zeros((b, h, w, 3 * dt), dtype=jnp.float32), spatial],
        axis=-1).reshape(b * h, w, d)

    mtable = jnp.concatenate(
        [jnp.zeros((12, 2 * dt), dtype=jnp.float32), _month_table(dt),
         jnp.zeros((12, dt), dtype=jnp.float32)], axis=-1)

    months3 = timestamps[:, 1, :].astype(jnp.int32).reshape(b, 1, t)

    tokens4 = jnp.transpose(tokens, (0, 1, 3, 4, 2, 5)).reshape(b * h, t * c, w, d)
    out = _run(tokens4, a_table, s_table, months3, mtable)
    return jnp.transpose(out.reshape(b, h, t, c, w, d), (0, 1, 4, 2, 3, 5))
